# baseline (speedup 1.0000x reference)
import functools

import jax
import jax.numpy as jnp
from jax import lax
from jax.experimental import pallas as pl
from jax.experimental.pallas import tpu as pltpu
from jax.experimental.pallas import tpu_sc as plsc

SCALES = (2, 4, 8, 16)
UNIT = 0.015
LIMIT = 64.0
OFFSET = -0.5 * UNIT * LIMIT

BN = 512
CH = 64


def _scale_topk(pb, pxyz, pp, v_ref, scale, bn):
    m = v_ref.shape[1]
    vs = UNIT * scale
    half = 0.5 * vs
    vb = v_ref[0:1, :]
    vx = (v_ref[1:2, :] * vs + OFFSET) + half
    vy = (v_ref[2:3, :] * vs + OFFSET) + half
    vz = (v_ref[3:4, :] * vs + OFFSET) + half
    qq = vx * vx + vy * vy + vz * vz
    vmat = jnp.concatenate([vx, vy, vz], axis=0).astype(jnp.bfloat16) * 2

    dot2 = jnp.dot(pxyz.astype(jnp.bfloat16), vmat,
                   preferred_element_type=jnp.float32)
    d = (pp + qq) - dot2
    d = jnp.maximum(d, 0.0)
    d = jnp.where(pb == vb, d, jnp.float32(1e10))

    iota = lax.broadcasted_iota(jnp.int32, (bn, m), 1).astype(jnp.float32)
    amins, rs = [], []
    for k in range(3):
        mk = jnp.min(d, axis=1, keepdims=True)
        amin = jnp.min(jnp.where(d == mk, iota, jnp.float32(m)),
                       axis=1, keepdims=True)
        amins.append(amin)
        rs.append(1.0 / (mk + 1e-8))
        if k < 2:
            d = jnp.where(iota == amin, jnp.float32(1e30), d)
    inorm = 1.0 / (rs[0] + rs[1] + rs[2])
    idx8 = jnp.concatenate(
        amins + [amins[0] * 0] * 5, axis=1)
    w48 = jnp.concatenate(
        [jnp.broadcast_to(r * inorm, (bn, 16)) for r in rs], axis=1)
    return idx8, w48


def _topk_kernel(pts_ref, v1, v2, v3, v4, oi1, ow1, oi2, ow2, oi3, ow3,
                 oi4, ow4, *, bn):
    pb = pts_ref[:, 0:1]
    px = pts_ref[:, 1:2]
    py = pts_ref[:, 2:3]
    pz = pts_ref[:, 3:4]
    pxyz = pts_ref[:, 1:4]
    pp = px * px + py * py + pz * pz
    for v_ref, oi, ow, scale in ((v1, oi1, ow1, SCALES[0]),
                                 (v2, oi2, ow2, SCALES[1]),
                                 (v3, oi3, ow3, SCALES[2]),
                                 (v4, oi4, ow4, SCALES[3])):
        idx8, w48 = _scale_topk(pb, pxyz, pp, v_ref, scale, bn)
        oi[...] = idx8
        ow[...] = w48


def _tc_topk(pts4, voxes, n):
    grid = (n // BN,)
    in_specs = [pl.BlockSpec((BN, 4), lambda i: (i, 0))]
    for v in voxes:
        in_specs.append(pl.BlockSpec(v.shape, lambda i: (0, 0)))
    out_specs, out_shape = [], []
    for _ in range(4):
        out_specs.append(pl.BlockSpec((BN, 8), lambda i: (i, 0)))
        out_specs.append(pl.BlockSpec((BN, 48), lambda i: (i, 0)))
        out_shape.append(jax.ShapeDtypeStruct((n, 8), jnp.float32))
        out_shape.append(jax.ShapeDtypeStruct((n, 48), jnp.float32))
    return pl.pallas_call(
        functools.partial(_topk_kernel, bn=BN),
        grid=grid,
        in_specs=in_specs,
        out_specs=out_specs,
        out_shape=out_shape,
    )(pts4, *voxes)


def _sc_interpolate(idxs, wts, feats, n):
    info = plsc.get_sparse_core_info()
    nw = info.num_cores * info.num_subcores
    per_w = n // nw
    cs = [f.shape[1] for f in feats]
    fpad = [jnp.pad(f, ((0, 0), (0, 128 - f.shape[1]))) for f in feats]

    scratch = [pltpu.VMEM((CH,), jnp.int32)]
    scratch += [pltpu.VMEM((CH, 48), jnp.float32)]
    scratch += [pltpu.VMEM((CH, 128), jnp.float32)] * 3
    for c in sorted(set(cs)):
        scratch += [pltpu.VMEM((CH, c), jnp.float32)]
    scratch.append(pltpu.SemaphoreType.DMA)

    mesh = plsc.VectorSubcoreMesh(core_axis_name="c", subcore_axis_name="s")

    @functools.partial(
        pl.kernel, mesh=mesh,
        out_type=[jax.ShapeDtypeStruct((n, c), jnp.float32) for c in cs],
        scratch_types=scratch,
    )
    def k(i1, i2, i3, i4, w1, w2, w3, w4, f1, f2, f3, f4,
          o1, o2, o3, o4, *scr):
        wid = lax.axis_index("s") * info.num_cores + lax.axis_index("c")
        base = wid * per_w
        idx_v = scr[0]
        w_v = scr[1]
        rows = scr[2:5]
        outs_v = {c: scr[5 + i] for i, c in enumerate(sorted(set(cs)))}
        sem = scr[-1]

        def do_scale(i_hbm, w_hbm, f_hbm, o_hbm, c):
            out_v = outs_v[c]

            def chunk_body(j, carry):
                pb0 = base + j * CH
                for kk in range(3):
                    pltpu.sync_copy(i_hbm.at[pl.ds(kk * n + pb0, CH)], idx_v)
                    pltpu.async_copy(f_hbm.at[idx_v], rows[kk], sem).wait()
                pltpu.sync_copy(w_hbm.at[pl.ds(pb0, CH)], w_v)

                for i in range(CH):
                    w0 = w_v[i, pl.ds(0, 16)]
                    w1_ = w_v[i, pl.ds(16, 16)]
                    w2_ = w_v[i, pl.ds(32, 16)]
                    for cc in range(c // 16):
                        sl = pl.ds(cc * 16, 16)
                        out_v[i, sl] = (w0 * rows[0][i, sl]
                                        + w1_ * rows[1][i, sl]
                                        + w2_ * rows[2][i, sl])
                pltpu.sync_copy(out_v, o_hbm.at[pl.ds(pb0, CH)])
                return carry

            lax.fori_loop(0, per_w // CH, chunk_body, 0)

        for i_hbm, w_hbm, f_hbm, o_hbm, c in ((i1, w1, f1, o1, cs[0]),
                                              (i2, w2, f2, o2, cs[1]),
                                              (i3, w3, f3, o3, cs[2]),
                                              (i4, w4, f4, o4, cs[3])):
            do_scale(i_hbm, w_hbm, f_hbm, o_hbm, c)

    return k(*idxs, *wts, *fpad)


@jax.jit
def kernel(points, batch_ids, feats1_features, feats1_indices,
           feats2_features, feats2_indices, feats3_features, feats3_indices,
           feats4_features, feats4_indices):
    n = points.shape[0]
    pts4 = jnp.concatenate(
        [batch_ids.reshape(-1, 1).astype(jnp.float32), points], axis=1)
    voxes = [jnp.transpose(ii).astype(jnp.float32)
             for ii in (feats1_indices, feats2_indices, feats3_indices,
                        feats4_indices)]
    feats = [feats1_features, feats2_features, feats3_features,
             feats4_features]

    packed = _tc_topk(pts4, voxes, n)
    idxs = [jnp.transpose(packed[2 * s][:, 0:3]).astype(jnp.int32).reshape(-1)
            for s in range(4)]
    wts = [packed[2 * s + 1] for s in range(4)]
    outs = _sc_interpolate(idxs, wts, feats, n)
    return jnp.concatenate(outs, axis=1)


# speedup vs baseline: 1.7026x; 1.7026x over previous
import functools

import jax
import jax.numpy as jnp
from jax import lax
from jax.experimental import pallas as pl

SCALES = (2, 4, 8, 16)
UNIT = 0.015
LIMIT = 64.0
OFFSET = -0.5 * UNIT * LIMIT

BN = 512


def _scale_body(pb, pxyz, pp, v_ref, f_ref, scale, bn):
    m = v_ref.shape[1]
    vs = UNIT * scale
    half = 0.5 * vs
    vb = v_ref[0:1, :]
    vx = (v_ref[1:2, :] * vs + OFFSET) + half
    vy = (v_ref[2:3, :] * vs + OFFSET) + half
    vz = (v_ref[3:4, :] * vs + OFFSET) + half
    qq = vx * vx + vy * vy + vz * vz
    vmat = jnp.concatenate([vx, vy, vz], axis=0).astype(jnp.bfloat16) * 2

    dot2 = jnp.dot(pxyz.astype(jnp.bfloat16), vmat,
                   preferred_element_type=jnp.float32)
    d = (pp + qq) - dot2
    d = jnp.maximum(d, 0.0)
    d = jnp.where(pb == vb, d, jnp.float32(1e10))

    iota = lax.broadcasted_iota(jnp.int32, (bn, m), 1).astype(jnp.float32)
    wu = jnp.zeros((bn, m), jnp.float32)
    norm = jnp.zeros((bn, 1), jnp.float32)
    for k in range(3):
        mk = jnp.min(d, axis=1, keepdims=True)
        amin = jnp.min(jnp.where(d == mk, iota, jnp.float32(m)),
                       axis=1, keepdims=True)
        onehot = iota == amin
        rk = 1.0 / (mk + 1e-8)
        wu = jnp.where(onehot, jnp.broadcast_to(rk, (bn, m)), wu)
        norm = norm + rk
        if k < 2:
            d = jnp.where(onehot, jnp.float32(1e30), d)
    w = wu * (1.0 / norm)
    return jnp.dot(w, f_ref[...], preferred_element_type=jnp.float32,
                   precision=lax.Precision.DEFAULT)


def _fused_kernel(pts_ref, v1, f1, v2, f2, v3, f3, v4, f4,
                  o1, o2, o3, o4, *, bn):
    pb = pts_ref[:, 0:1]
    px = pts_ref[:, 1:2]
    py = pts_ref[:, 2:3]
    pz = pts_ref[:, 3:4]
    pxyz = pts_ref[:, 1:4]
    pp = px * px + py * py + pz * pz
    for v_ref, f_ref, o_ref, scale in ((v1, f1, o1, SCALES[0]),
                                       (v2, f2, o2, SCALES[1]),
                                       (v3, f3, o3, SCALES[2]),
                                       (v4, f4, o4, SCALES[3])):
        o_ref[...] = _scale_body(pb, pxyz, pp, v_ref, f_ref, scale, bn)


@jax.jit
def kernel(points, batch_ids, feats1_features, feats1_indices,
           feats2_features, feats2_indices, feats3_features, feats3_indices,
           feats4_features, feats4_indices):
    n = points.shape[0]
    pts4 = jnp.concatenate(
        [batch_ids.reshape(-1, 1).astype(jnp.float32), points], axis=1)
    voxes = [jnp.transpose(ii).astype(jnp.float32)
             for ii in (feats1_indices, feats2_indices, feats3_indices,
                        feats4_indices)]
    feats = [feats1_features, feats2_features, feats3_features,
             feats4_features]
    cs = [f.shape[1] for f in feats]
    ms = [v.shape[1] for v in voxes]

    grid = (n // BN,)
    in_specs = [pl.BlockSpec((BN, 4), lambda i: (i, 0))]
    for v, f in zip(voxes, feats):
        in_specs.append(pl.BlockSpec(v.shape, lambda i: (0, 0)))
        in_specs.append(pl.BlockSpec(f.shape, lambda i: (0, 0)))
    out_specs = [pl.BlockSpec((BN, C), lambda i: (i, 0)) for C in cs]
    out_shape = [jax.ShapeDtypeStruct((n, C), jnp.float32) for C in cs]

    args = [pts4]
    for v, f in zip(voxes, feats):
        args.extend((v, f))
    outs = pl.pallas_call(
        functools.partial(_fused_kernel, bn=BN),
        grid=grid,
        in_specs=in_specs,
        out_specs=out_specs,
        out_shape=out_shape,
    )(*args)
    return jnp.concatenate(outs, axis=1)


# speedup vs baseline: 1.7424x; 1.0234x over previous
import functools

import jax
import jax.numpy as jnp
from jax import lax
from jax.experimental import pallas as pl

SCALES = (2, 4, 8, 16)
UNIT = 0.015
LIMIT = 64.0
OFFSET = -0.5 * UNIT * LIMIT

BN = 512


def _scale_body(pb, pxyz, pp, v_ref, f_ref, scale, bn):
    m = v_ref.shape[1]
    vs = UNIT * scale
    half = 0.5 * vs
    vb = v_ref[0:1, :]
    vx = (v_ref[1:2, :] * vs + OFFSET) + half
    vy = (v_ref[2:3, :] * vs + OFFSET) + half
    vz = (v_ref[3:4, :] * vs + OFFSET) + half
    qq = vx * vx + vy * vy + vz * vz
    vmat = jnp.concatenate([vx, vy, vz], axis=0).astype(jnp.bfloat16) * 2

    dot2 = jnp.dot(pxyz.astype(jnp.bfloat16), vmat,
                   preferred_element_type=jnp.float32)
    d = (pp + qq) - dot2
    d = jnp.maximum(d, 0.0)
    d = jnp.where(pb == vb, d, jnp.float32(1e10))

    iota = lax.broadcasted_iota(jnp.int32, (bn, m), 1).astype(jnp.float32)
    wu = jnp.zeros((bn, m), jnp.float32)
    norm = jnp.zeros((bn, 1), jnp.float32)
    for k in range(3):
        mk = jnp.min(d, axis=1, keepdims=True)
        amin = jnp.min(jnp.where(d == mk, iota, jnp.float32(m)),
                       axis=1, keepdims=True)
        onehot = iota == amin
        rk = 1.0 / (mk + 1e-8)
        wu = jnp.where(onehot, jnp.broadcast_to(rk, (bn, m)), wu)
        norm = norm + rk
        if k < 2:
            d = jnp.where(onehot, jnp.float32(1e30), d)
    out = jnp.dot(wu, f_ref[...], preferred_element_type=jnp.float32,
                  precision=lax.Precision.DEFAULT)
    return out * (1.0 / norm)


def _fused_kernel(pts_ref, v1, f1, v2, f2, v3, f3, v4, f4,
                  o1, o2, o3, o4, *, bn):
    pb = pts_ref[:, 0:1]
    px = pts_ref[:, 1:2]
    py = pts_ref[:, 2:3]
    pz = pts_ref[:, 3:4]
    pxyz = pts_ref[:, 1:4]
    pp = px * px + py * py + pz * pz
    for v_ref, f_ref, o_ref, scale in ((v1, f1, o1, SCALES[0]),
                                       (v2, f2, o2, SCALES[1]),
                                       (v3, f3, o3, SCALES[2]),
                                       (v4, f4, o4, SCALES[3])):
        o_ref[...] = _scale_body(pb, pxyz, pp, v_ref, f_ref, scale, bn)


@jax.jit
def kernel(points, batch_ids, feats1_features, feats1_indices,
           feats2_features, feats2_indices, feats3_features, feats3_indices,
           feats4_features, feats4_indices):
    n = points.shape[0]
    pts4 = jnp.concatenate(
        [batch_ids.reshape(-1, 1).astype(jnp.float32), points], axis=1)
    voxes = [jnp.transpose(ii).astype(jnp.float32)
             for ii in (feats1_indices, feats2_indices, feats3_indices,
                        feats4_indices)]
    feats = [feats1_features, feats2_features, feats3_features,
             feats4_features]
    cs = [f.shape[1] for f in feats]
    ms = [v.shape[1] for v in voxes]

    grid = (n // BN,)
    in_specs = [pl.BlockSpec((BN, 4), lambda i: (i, 0))]
    for v, f in zip(voxes, feats):
        in_specs.append(pl.BlockSpec(v.shape, lambda i: (0, 0)))
        in_specs.append(pl.BlockSpec(f.shape, lambda i: (0, 0)))
    out_specs = [pl.BlockSpec((BN, C), lambda i: (i, 0)) for C in cs]
    out_shape = [jax.ShapeDtypeStruct((n, C), jnp.float32) for C in cs]

    args = [pts4]
    for v, f in zip(voxes, feats):
        args.extend((v, f))
    outs = pl.pallas_call(
        functools.partial(_fused_kernel, bn=BN),
        grid=grid,
        in_specs=in_specs,
        out_specs=out_specs,
        out_shape=out_shape,
    )(*args)
    return jnp.concatenate(outs, axis=1)
